# baseline (speedup 1.0000x reference)
import functools

import jax
import jax.numpy as jnp
from jax import lax
from jax.experimental import pallas as pl
from jax.experimental.pallas import tpu as pltpu
from jax.experimental.pallas import tpu_sc as plsc

_B = 1024
_NBR = 16
_DIM = 128
_NW = 32
_BPW = _B // _NW
_ROWS = _BPW * _NBR
_NCHUNK = 4
_CROWS = _ROWS // _NCHUNK
_LANES = 16
_ADJ_PACK = 128 // _NBR


def _sc_body(v_hbm, adj_hbm, ent_hbm, out_hbm, vidx, vrow, nbr, flat, rows,
             outbuf, sem):
    wid = lax.axis_index("s") * 2 + lax.axis_index("c")
    base = wid * _BPW

    pltpu.sync_copy(v_hbm.at[pl.ds(base, _BPW)], vidx)

    for t in range(_BPW // _LANES):
        sl = pl.ds(t * _LANES, _LANES)
        vrow[sl] = vidx[sl] >> 3

    pltpu.async_copy(adj_hbm.at[vrow], nbr, sem).wait()

    for t in range(_BPW // _LANES):
        v16 = vidx[pl.ds(t * _LANES, _LANES)]
        for u in range(_LANES):
            k = t * _LANES + u
            off = (v16[u] & (_ADJ_PACK - 1)) * _NBR
            nbrk = nbr[k, pl.ds(off, _NBR)]
            flat[k // (_CROWS // _NBR),
                 pl.ds((k % (_CROWS // _NBR)) * _NBR, _NBR)] = nbrk

    copies = []
    for c in range(_NCHUNK):
        copies.append(
            pltpu.async_copy(
                ent_hbm.at[flat.at[c]], rows.at[pl.ds(c * _CROWS, _CROWS)], sem
            )
        )
    for cp in copies:
        cp.wait()

    def body(i, carry):
        r0 = i * _NBR
        for c in range(_DIM // _LANES):
            sl = pl.ds(c * _LANES, _LANES)
            acc = rows[r0, sl]
            for j in range(1, _NBR):
                acc = acc + rows[r0 + j, sl]
            outbuf[i, sl] = acc * (1.0 / _NBR)
        return carry

    lax.fori_loop(0, _BPW, body, 0)

    pltpu.sync_copy(outbuf, out_hbm.at[pl.ds(base, _BPW)])


@jax.jit
def kernel(v, adj_ent, ent):
    v = v.astype(jnp.int32)
    adj_packed = adj_ent.astype(jnp.int32).reshape(-1, 128)
    ent = ent.astype(jnp.float32)

    mesh = plsc.VectorSubcoreMesh(core_axis_name="c", subcore_axis_name="s")
    run = functools.partial(
        pl.kernel,
        mesh=mesh,
        out_type=jax.ShapeDtypeStruct((_B, _DIM), jnp.float32),
        scratch_types=[
            pltpu.VMEM((_BPW,), jnp.int32),
            pltpu.VMEM((_BPW,), jnp.int32),
            pltpu.VMEM((_BPW, 128), jnp.int32),
            pltpu.VMEM((_NCHUNK, _CROWS), jnp.int32),
            pltpu.VMEM((_ROWS, _DIM), jnp.float32),
            pltpu.VMEM((_BPW, _DIM), jnp.float32),
            pltpu.SemaphoreType.DMA,
        ],
    )(_sc_body)
    return run(v, adj_packed, ent)


# speedup vs baseline: 1.2957x; 1.2957x over previous
import functools

import jax
import jax.numpy as jnp
from jax import lax
from jax.experimental import pallas as pl
from jax.experimental.pallas import tpu as pltpu
from jax.experimental.pallas import tpu_sc as plsc

_B = 1024
_NBR = 16
_DIM = 128
_NW = 32
_BPW = _B // _NW
_ROWS = _BPW * _NBR
_NCHUNK = 4
_CROWS = _ROWS // _NCHUNK
_SPC = _CROWS // _NBR
_LANES = 16


def _sc_body(v_hbm, adj_hbm, ent_hbm, out_hbm, vidx, flat, rows, outbuf,
             sema, sem0, sem1, sem2, sem3):
    wid = lax.axis_index("s") * 2 + lax.axis_index("c")
    base = wid * _BPW

    pltpu.sync_copy(v_hbm.at[pl.ds(base, _BPW)], vidx)

    adj_copies = []
    for t in range(_BPW // _LANES):
        v16 = vidx[pl.ds(t * _LANES, _LANES)]
        for u in range(_LANES):
            k = t * _LANES + u
            adj_copies.append(
                pltpu.async_copy(
                    adj_hbm.at[v16[u]],
                    flat.at[k // _SPC, pl.ds((k % _SPC) * _NBR, _NBR)],
                    sema,
                )
            )
    for cp in adj_copies:
        cp.wait()

    sems = [sem0, sem1, sem2, sem3]
    copies = []
    for c in range(_NCHUNK):
        copies.append(
            pltpu.async_copy(
                ent_hbm.at[flat.at[c]], rows.at[pl.ds(c * _CROWS, _CROWS)],
                sems[c],
            )
        )

    for c in range(_NCHUNK):
        copies[c].wait()

        def body(i, carry):
            r0 = i * _NBR
            for d in range(_DIM // _LANES):
                sl = pl.ds(d * _LANES, _LANES)
                acc = rows[r0, sl]
                for j in range(1, _NBR):
                    acc = acc + rows[r0 + j, sl]
                outbuf[i, sl] = acc * (1.0 / _NBR)
            return carry

        lax.fori_loop(c * _SPC, (c + 1) * _SPC, body, 0)

    pltpu.sync_copy(outbuf, out_hbm.at[pl.ds(base, _BPW)])


@jax.jit
def kernel(v, adj_ent, ent):
    v = v.astype(jnp.int32)
    adj_ent = adj_ent.astype(jnp.int32)
    ent = ent.astype(jnp.float32)

    mesh = plsc.VectorSubcoreMesh(core_axis_name="c", subcore_axis_name="s")
    run = functools.partial(
        pl.kernel,
        mesh=mesh,
        out_type=jax.ShapeDtypeStruct((_B, _DIM), jnp.float32),
        scratch_types=[
            pltpu.VMEM((_BPW,), jnp.int32),
            pltpu.VMEM((_NCHUNK, _CROWS), jnp.int32),
            pltpu.VMEM((_ROWS, _DIM), jnp.float32),
            pltpu.VMEM((_BPW, _DIM), jnp.float32),
            pltpu.SemaphoreType.DMA,
            pltpu.SemaphoreType.DMA,
            pltpu.SemaphoreType.DMA,
            pltpu.SemaphoreType.DMA,
            pltpu.SemaphoreType.DMA,
        ],
    )(_sc_body)
    return run(v, adj_ent, ent)


# speedup vs baseline: 1.9050x; 1.4703x over previous
import functools

import jax
import jax.numpy as jnp
from jax import lax
from jax.experimental import pallas as pl
from jax.experimental.pallas import tpu as pltpu
from jax.experimental.pallas import tpu_sc as plsc

_B = 1024
_NBR = 16
_DIM = 128
_NW = 32
_BPW = _B // _NW
_ROWS = _BPW * _NBR
_NCHUNK = 4
_CROWS = _ROWS // _NCHUNK
_SPC = _CROWS // _NBR
_WAVE = 16
_LANES = 16


def _sc_body(v_hbm, adjT_hbm, ent_hbm, out_hbm, vidx, blocks, flat, rows,
             outbuf, sema, sem0, sem1, sem2, sem3):
    wid = lax.axis_index("s") * 2 + lax.axis_index("c")
    base = wid * _BPW

    pltpu.sync_copy(v_hbm.at[pl.ds(base, _BPW)], vidx)

    sems = [sem0, sem1, sem2, sem3]
    lane = lax.iota(jnp.int32, _LANES)
    ent_copies = []

    def fire_wave(w, vks):
        copies = []
        for s in range(_WAVE):
            colbase = (vks[s] >> 7) * 128
            copies.append(
                pltpu.async_copy(
                    adjT_hbm.at[:, pl.ds(colbase, 128)], blocks.at[s], sema
                )
            )
        return copies

    def extract_wave(w, vks):
        for s in range(_WAVE):
            k = w * _WAVE + s
            c0 = vks[s] & 127
            cbase = c0 & 112
            l0 = jnp.full((_LANES,), c0 & 15, jnp.int32)
            res = jnp.zeros((_LANES,), jnp.int32)
            for j in range(_NBR):
                bv = blocks[s, j, pl.ds(cbase, _LANES)]
                t = bv.at[l0].get(mode="promise_in_bounds")
                res = jnp.where(lane == j, t, res)
            flat[k // _SPC, pl.ds((k % _SPC) * _NBR, _NBR)] = res
            if (k + 1) % _SPC == 0:
                c = k // _SPC
                ent_copies.append(
                    pltpu.async_copy(
                        ent_hbm.at[flat.at[c]],
                        rows.at[pl.ds(c * _CROWS, _CROWS)],
                        sems[c],
                    )
                )

    v16a = vidx[pl.ds(0, _LANES)]
    vks_a = [v16a[s] for s in range(_WAVE)]
    wave_copies = fire_wave(0, vks_a)
    v16b = vidx[pl.ds(_LANES, _LANES)]
    vks_b = [v16b[s] for s in range(_WAVE)]
    for cp in wave_copies:
        cp.wait()
    extract_wave(0, vks_a)

    wave_copies = fire_wave(1, vks_b)
    for cp in wave_copies:
        cp.wait()
    extract_wave(1, vks_b)

    for c in range(_NCHUNK):
        ent_copies[c].wait()

        def body(i, carry):
            r0 = i * _NBR
            for d in range(_DIM // _LANES):
                sl = pl.ds(d * _LANES, _LANES)
                acc = rows[r0, sl]
                for j in range(1, _NBR):
                    acc = acc + rows[r0 + j, sl]
                outbuf[i, sl] = acc * (1.0 / _NBR)
            return carry

        lax.fori_loop(c * _SPC, (c + 1) * _SPC, body, 0)

    pltpu.sync_copy(outbuf, out_hbm.at[pl.ds(base, _BPW)])


@jax.jit
def kernel(v, adj_ent, ent):
    v = v.astype(jnp.int32)
    adjT = adj_ent.astype(jnp.int32).T
    ent = ent.astype(jnp.float32)

    mesh = plsc.VectorSubcoreMesh(core_axis_name="c", subcore_axis_name="s")
    run = functools.partial(
        pl.kernel,
        mesh=mesh,
        out_type=jax.ShapeDtypeStruct((_B, _DIM), jnp.float32),
        scratch_types=[
            pltpu.VMEM((_BPW,), jnp.int32),
            pltpu.VMEM((_WAVE, _NBR, 128), jnp.int32),
            pltpu.VMEM((_NCHUNK, _CROWS), jnp.int32),
            pltpu.VMEM((_ROWS, _DIM), jnp.float32),
            pltpu.VMEM((_BPW, _DIM), jnp.float32),
            pltpu.SemaphoreType.DMA,
            pltpu.SemaphoreType.DMA,
            pltpu.SemaphoreType.DMA,
            pltpu.SemaphoreType.DMA,
            pltpu.SemaphoreType.DMA,
        ],
    )(_sc_body)
    return run(v, adjT, ent)


# speedup vs baseline: 1.9658x; 1.0319x over previous
import functools

import jax
import jax.numpy as jnp
from jax import lax
from jax.experimental import pallas as pl
from jax.experimental.pallas import tpu as pltpu
from jax.experimental.pallas import tpu_sc as plsc

_B = 1024
_NBR = 16
_DIM = 128
_NW = 32
_BPW = _B // _NW
_ROWS = _BPW * _NBR
_NCHUNK = 4
_CROWS = _ROWS // _NCHUNK
_SPC = _CROWS // _NBR
_WAVE = 16
_LANES = 16


def _sc_body(v_hbm, adjT_hbm, ent_hbm, out_hbm, vidx, blocks, flat, rows,
             outbuf, vsm, sema, sem0, sem1, sem2, sem3):
    wid = lax.axis_index("s") * 2 + lax.axis_index("c")
    base = wid * _BPW

    pltpu.sync_copy(v_hbm.at[pl.ds(base, _BPW)], vidx)
    for t in range(_BPW // _LANES):
        v16 = vidx[pl.ds(t * _LANES, _LANES)]
        for u in range(_LANES):
            vsm[t * _LANES + u] = v16[u]

    sems = [sem0, sem1, sem2, sem3]
    lane = lax.iota(jnp.int32, _LANES)

    def fire_wave(w):
        def fbody(u, carry):
            colbase = (vsm[w * _WAVE + u] >> 7) * 128
            pltpu.async_copy(
                adjT_hbm.at[:, pl.ds(colbase, 128)], blocks.at[u], sema
            )
            return carry
        lax.fori_loop(0, _WAVE, fbody, 0)

    def drain_wave():
        def dbody(u, carry):
            pltpu.make_async_copy(
                adjT_hbm.at[:, pl.ds(0, 128)], blocks.at[0], sema
            ).wait()
            return carry
        lax.fori_loop(0, _WAVE, dbody, 0)

    def extract_wave(w):
        def ebody(u, carry):
            vk = vsm[w * _WAVE + u]
            c0 = vk & 127
            cbase = c0 & 112
            l0 = jnp.full((_LANES,), c0 & 15, jnp.int32)
            res = jnp.zeros((_LANES,), jnp.int32)
            for j in range(_NBR):
                bv = blocks[u, j, pl.ds(cbase, _LANES)]
                t = bv.at[l0].get(mode="promise_in_bounds")
                res = jnp.where(lane == j, t, res)
            k = w * _WAVE + u
            flat[k // _SPC, pl.ds((k % _SPC) * _NBR, _NBR)] = res
            return carry
        lax.fori_loop(0, _WAVE, ebody, 0)

    def fire_ent(c):
        return pltpu.async_copy(
            ent_hbm.at[flat.at[c]], rows.at[pl.ds(c * _CROWS, _CROWS)],
            sems[c],
        )

    fire_wave(0)
    drain_wave()
    extract_wave(0)
    ent_copies = [fire_ent(0), fire_ent(1)]

    fire_wave(1)
    drain_wave()
    extract_wave(1)
    ent_copies += [fire_ent(2), fire_ent(3)]

    for c in range(_NCHUNK):
        ent_copies[c].wait()

        def body(i, carry):
            r0 = i * _NBR
            for d in range(_DIM // _LANES):
                sl = pl.ds(d * _LANES, _LANES)
                acc = rows[r0, sl]
                for j in range(1, _NBR):
                    acc = acc + rows[r0 + j, sl]
                outbuf[i, sl] = acc * (1.0 / _NBR)
            return carry

        lax.fori_loop(c * _SPC, (c + 1) * _SPC, body, 0)

    pltpu.sync_copy(outbuf, out_hbm.at[pl.ds(base, _BPW)])


@jax.jit
def kernel(v, adj_ent, ent):
    v = v.astype(jnp.int32)
    adjT = adj_ent.astype(jnp.int32).T
    ent = ent.astype(jnp.float32)

    mesh = plsc.VectorSubcoreMesh(core_axis_name="c", subcore_axis_name="s")
    run = functools.partial(
        pl.kernel,
        mesh=mesh,
        out_type=jax.ShapeDtypeStruct((_B, _DIM), jnp.float32),
        scratch_types=[
            pltpu.VMEM((_BPW,), jnp.int32),
            pltpu.VMEM((_WAVE, _NBR, 128), jnp.int32),
            pltpu.VMEM((_NCHUNK, _CROWS), jnp.int32),
            pltpu.VMEM((_ROWS, _DIM), jnp.float32),
            pltpu.VMEM((_BPW, _DIM), jnp.float32),
            pltpu.SMEM((_BPW,), jnp.int32),
            pltpu.SemaphoreType.DMA,
            pltpu.SemaphoreType.DMA,
            pltpu.SemaphoreType.DMA,
            pltpu.SemaphoreType.DMA,
            pltpu.SemaphoreType.DMA,
        ],
    )(_sc_body)
    return run(v, adjT, ent)


# speedup vs baseline: 1.9918x; 1.0132x over previous
import functools

import jax
import jax.numpy as jnp
from jax import lax
from jax.experimental import pallas as pl
from jax.experimental.pallas import tpu as pltpu
from jax.experimental.pallas import tpu_sc as plsc

_B = 1024
_NBR = 16
_DIM = 128
_NW = 32
_BPW = _B // _NW
_ROWS = _BPW * _NBR
_NCHUNK = 4
_CROWS = _ROWS // _NCHUNK
_SPC = _CROWS // _NBR
_WAVE = 16
_LANES = 16


def _sc_body(v_hbm, adjT_hbm, ent_hbm, out_hbm, vidx, blocks, flat, rows,
             outbuf, vsm, sema, semb, sem0, sem1, sem2, sem3):
    wid = lax.axis_index("s") * 2 + lax.axis_index("c")
    base = wid * _BPW

    pltpu.sync_copy(v_hbm.at[pl.ds(base, _BPW)], vidx)
    for t in range(_BPW // _LANES):
        v16 = vidx[pl.ds(t * _LANES, _LANES)]
        for u in range(_LANES):
            vsm[t * _LANES + u] = v16[u]

    lane = lax.iota(jnp.int32, _LANES)
    wave_sems = [sema, semb]
    ent_sems = [sem0, sem1, sem2, sem3]

    def fire_wave(w):
        def fbody(u, carry):
            colbase = (vsm[w * _WAVE + u] >> 7) * 128
            pltpu.async_copy(
                adjT_hbm.at[:, pl.ds(colbase, 128)],
                blocks.at[w * _WAVE + u], wave_sems[w],
            )
            return carry
        lax.fori_loop(0, _WAVE, fbody, 0)

    def drain_wave(w):
        def dbody(u, carry):
            pltpu.make_async_copy(
                adjT_hbm.at[:, pl.ds(0, 128)], blocks.at[0], wave_sems[w]
            ).wait()
            return carry
        lax.fori_loop(0, _WAVE, dbody, 0)

    def extract_wave(w):
        def ebody(u, carry):
            k = w * _WAVE + u
            vk = vsm[k]
            c0 = vk & 127
            cbase = c0 & 112
            l0 = jnp.full((_LANES,), c0 & 15, jnp.int32)
            res = jnp.zeros((_LANES,), jnp.int32)
            for j in range(_NBR):
                bv = blocks[k, j, pl.ds(cbase, _LANES)]
                t = bv.at[l0].get(mode="promise_in_bounds")
                res = jnp.where(lane == j, t, res)
            flat[k // _SPC, pl.ds((k % _SPC) * _NBR, _NBR)] = res
            return carry
        lax.fori_loop(0, _WAVE, ebody, 0)

    def fire_ent(c):
        return pltpu.async_copy(
            ent_hbm.at[flat.at[c]],
            rows.at[pl.ds((c % 2) * _CROWS, _CROWS)],
            ent_sems[c],
        )

    def reduce_chunk(c):
        def body(i, carry):
            r0 = (c % 2) * _CROWS + i * _NBR
            for d in range(_DIM // _LANES):
                sl = pl.ds(d * _LANES, _LANES)
                acc = rows[r0, sl]
                for j in range(1, _NBR):
                    acc = acc + rows[r0 + j, sl]
                outbuf[c * _SPC + i, sl] = acc * (1.0 / _NBR)
            return carry
        lax.fori_loop(0, _SPC, body, 0)

    fire_wave(0)
    fire_wave(1)
    drain_wave(0)
    extract_wave(0)
    ent_copies = [fire_ent(0), fire_ent(1)]
    drain_wave(1)
    extract_wave(1)

    ent_copies[0].wait()
    reduce_chunk(0)
    ent_copies.append(fire_ent(2))
    ent_copies[1].wait()
    reduce_chunk(1)
    ent_copies.append(fire_ent(3))
    ent_copies[2].wait()
    reduce_chunk(2)
    ent_copies[3].wait()
    reduce_chunk(3)

    pltpu.sync_copy(outbuf, out_hbm.at[pl.ds(base, _BPW)])


@jax.jit
def kernel(v, adj_ent, ent):
    v = v.astype(jnp.int32)
    adjT = adj_ent.astype(jnp.int32).T
    ent = ent.astype(jnp.float32)

    mesh = plsc.VectorSubcoreMesh(core_axis_name="c", subcore_axis_name="s")
    run = functools.partial(
        pl.kernel,
        mesh=mesh,
        out_type=jax.ShapeDtypeStruct((_B, _DIM), jnp.float32),
        scratch_types=[
            pltpu.VMEM((_BPW,), jnp.int32),
            pltpu.VMEM((_BPW, _NBR, 128), jnp.int32),
            pltpu.VMEM((_NCHUNK, _CROWS), jnp.int32),
            pltpu.VMEM((2 * _CROWS, _DIM), jnp.float32),
            pltpu.VMEM((_BPW, _DIM), jnp.float32),
            pltpu.SMEM((_BPW,), jnp.int32),
            pltpu.SemaphoreType.DMA,
            pltpu.SemaphoreType.DMA,
            pltpu.SemaphoreType.DMA,
            pltpu.SemaphoreType.DMA,
            pltpu.SemaphoreType.DMA,
            pltpu.SemaphoreType.DMA,
        ],
    )(_sc_body)
    return run(v, adjT, ent)
